# baseline (speedup 1.0000x reference)
import functools

import jax
import jax.numpy as jnp
from jax import lax
from jax.experimental import pallas as pl
from jax.experimental.pallas import tpu as pltpu
from jax.experimental.pallas import tpu_sc as plsc

CLS_DIM = 1000000
EMB_DIM = 64
BATCH = 16384

NUM_CORES = 2
NUM_SUBCORES = 16
NUM_WORKERS = NUM_CORES * NUM_SUBCORES
B_PER_W = BATCH // NUM_WORKERS
CHUNK = 128
NCHUNK = B_PER_W // CHUNK
LANES = 16

TBK = 512


NPACK = 977 * TBK


def _transpose_block(x1_ref, x2_ref, o_ref):
    o_ref[:, :EMB_DIM] = x1_ref[...].T
    o_ref[:, EMB_DIM:] = x2_ref[...].T


def _make_transpose():
    grid = NPACK // TBK
    return pl.pallas_call(
        _transpose_block,
        grid=(grid,),
        in_specs=[
            pl.BlockSpec((EMB_DIM, TBK), lambda i: (0, 2 * i)),
            pl.BlockSpec((EMB_DIM, TBK), lambda i: (0, 2 * i + 1)),
        ],
        out_specs=pl.BlockSpec((TBK, 2 * EMB_DIM), lambda i: (i, 0)),
        out_shape=jax.ShapeDtypeStruct((NPACK, 2 * EMB_DIM), jnp.float32),
    )


def _make_gather():
    mesh = plsc.VectorSubcoreMesh(core_axis_name="c", subcore_axis_name="s")

    @functools.partial(
        pl.kernel,
        mesh=mesh,
        out_type=jax.ShapeDtypeStruct((BATCH * EMB_DIM // 128, 128), jnp.float32),
        scratch_types=[
            pltpu.VMEM((NCHUNK, CHUNK), jnp.int32),
            pltpu.VMEM((B_PER_W, LANES), jnp.int32),
            pltpu.VMEM((2, CHUNK, 2 * EMB_DIM), jnp.float32),
            pltpu.VMEM((2, CHUNK * EMB_DIM // 128, 128), jnp.float32),
            pltpu.SemaphoreType.DMA,
            pltpu.SemaphoreType.DMA,
            pltpu.SemaphoreType.DMA,
        ],
    )
    def gather_kernel(pidx_hbm, half_hbm, table2_hbm, out_hbm,
                      pidx_v, half_v, pairbuf, outbuf, sem, hsem, osem):
        wid = lax.axis_index("s") * NUM_CORES + lax.axis_index("c")
        hcopy = pltpu.async_copy(half_hbm.at[wid], half_v, hsem)
        pltpu.sync_copy(pidx_hbm.at[wid], pidx_v)

        def fire(c):
            return pltpu.async_copy(
                table2_hbm.at[pidx_v.at[c]],
                pairbuf.at[c % 2],
                sem,
            )

        copies = [fire(0), fire(1)]
        hcopy.wait()

        def make_extract(c):
            def extract_row(j, _):
                m = half_v[c * CHUNK + j, :] == 1
                row = lax.shift_right_logical(j, 1)
                colbase = lax.bitwise_and(j, 1) * EMB_DIM
                for c4 in range(EMB_DIM // LANES):
                    a = pairbuf[c % 2, j, pl.ds(c4 * LANES, LANES)]
                    b = pairbuf[c % 2, j, pl.ds(EMB_DIM + c4 * LANES, LANES)]
                    outbuf[c % 2, row, pl.ds(colbase + c4 * LANES, LANES)] = (
                        jnp.where(m, b, a)
                    )
                return _

            return extract_row

        orows = CHUNK * EMB_DIM // 128
        obase = wid * B_PER_W * EMB_DIM // 128
        ocopies = []
        for c in range(NCHUNK):
            copies[c].wait()
            if c >= 2:
                ocopies[c - 2].wait()
            lax.fori_loop(0, CHUNK, make_extract(c), None)
            ocopies.append(
                pltpu.async_copy(
                    outbuf.at[c % 2],
                    out_hbm.at[pl.ds(pl.multiple_of(obase + c * orows, 8), orows)],
                    osem,
                )
            )
            if c + 2 < NCHUNK:
                copies.append(fire(c + 2))
        for c in range(NCHUNK - 2, NCHUNK):
            ocopies[c].wait()

    return gather_kernel


_transpose = _make_transpose()
_gather = _make_gather()


@jax.jit
def kernel(batch, table):
    idx = batch.astype(jnp.int32)
    pidx = (
        lax.shift_right_logical(idx, 10) * TBK + (idx & (TBK - 1))
    ).reshape(NUM_WORKERS, NCHUNK, CHUNK)
    half = jnp.broadcast_to(
        (lax.shift_right_logical(idx, 9) & 1)[:, None], (BATCH, LANES)
    ).reshape(NUM_WORKERS, B_PER_W, LANES)
    tableT = table.T
    table2 = _transpose(tableT, tableT)
    out = _gather(pidx, half, table2)
    return out.reshape(BATCH, EMB_DIM)


# speedup vs baseline: 1.8405x; 1.8405x over previous
import functools

import jax
import jax.numpy as jnp
from jax import lax
from jax.experimental import pallas as pl
from jax.experimental.pallas import tpu as pltpu
from jax.experimental.pallas import tpu_sc as plsc

CLS_DIM = 1000000
EMB_DIM = 64
BATCH = 16384

NUM_CORES = 2
NUM_SUBCORES = 16
NUM_WORKERS = NUM_CORES * NUM_SUBCORES
B_PER_W = BATCH // NUM_WORKERS
LANES = 16
NCHUNK = B_PER_W // LANES


def _make_kernel():
    mesh = plsc.VectorSubcoreMesh(core_axis_name="c", subcore_axis_name="s")

    @functools.partial(
        pl.kernel,
        mesh=mesh,
        out_type=jax.ShapeDtypeStruct((BATCH * EMB_DIM // 128, 128), jnp.float32),
        scratch_types=[
            pltpu.VMEM((B_PER_W,), jnp.int32),
            pltpu.VMEM((2, LANES, 8, EMB_DIM), jnp.float32),
            pltpu.VMEM((2, LANES * EMB_DIM // 128, 128), jnp.float32),
            pltpu.SemaphoreType.DMA,
            pltpu.SemaphoreType.DMA,
            pltpu.SemaphoreType.DMA,
            pltpu.SemaphoreType.DMA,
        ],
        compiler_params=pltpu.CompilerParams(needs_layout_passes=False),
    )
    def gather_kernel(idx_hbm, tab_hbm, out_hbm,
                      idx_v, slots, outbuf, sem0, sem1, osem0, osem1):
        wid = lax.axis_index("s") * NUM_CORES + lax.axis_index("c")
        pltpu.sync_copy(idx_hbm.at[wid], idx_v)
        lanes = lax.iota(jnp.int32, LANES)
        sems = [sem0, sem1]
        osems = [osem0, osem1]
        orows = LANES * EMB_DIM // 128
        obase = wid * B_PER_W * EMB_DIM // 128

        def scalar_at(base16, l):
            v = idx_v[pl.ds(base16, LANES)]
            return jnp.sum(jnp.where(lanes == l, v, 0))

        def fire(c, par):
            for l in range(LANES):
                i = scalar_at(c * LANES, l)
                base = pl.multiple_of(i - lax.bitwise_and(i, 7), 8)
                pltpu.async_copy(
                    tab_hbm.at[pl.ds(base, 8)], slots.at[par, l], sems[par]
                )

        def drain(d, par):
            for l in range(LANES):
                pltpu.make_async_copy(
                    tab_hbm.at[pl.ds(0, 8)], slots.at[par, l], sems[par]
                ).wait()
            for l in range(LANES):
                i = scalar_at(d * LANES, l)
                r = lax.bitwise_and(i, 7)
                row = l // 2
                colbase = (l % 2) * EMB_DIM
                for c4 in range(EMB_DIM // LANES):
                    outbuf[par, row, pl.ds(colbase + c4 * LANES, LANES)] = (
                        slots[par, l, r, pl.ds(c4 * LANES, LANES)]
                    )
            pltpu.async_copy(
                outbuf.at[par],
                out_hbm.at[pl.ds(pl.multiple_of(obase + d * orows, 8), orows)],
                osems[par],
            )

        def owait(par):
            pltpu.make_async_copy(
                out_hbm.at[pl.ds(0, orows)], outbuf.at[par], osems[par]
            ).wait()

        fire(0, 0)

        def body(c2, _):
            a = 2 * c2
            b = a + 1

            @pl.when(c2 >= 1)
            def _w0():
                owait(0)

            fire(b, 1)
            drain(a, 0)

            @pl.when(c2 >= 1)
            def _w1():
                owait(1)

            @pl.when(c2 < NCHUNK // 2 - 1)
            def _f1():
                fire(b + 1, 0)

            drain(b, 1)
            return _

        lax.fori_loop(0, NCHUNK // 2, body, None)
        owait(0)
        owait(1)

    return gather_kernel


_gather = _make_kernel()


@jax.jit
def kernel(batch, table):
    idx = batch.astype(jnp.int32).reshape(NUM_WORKERS, B_PER_W)
    out = _gather(idx, table)
    return out.reshape(BATCH, EMB_DIM)


# speedup vs baseline: 2.6205x; 1.4238x over previous
import functools

import jax
import jax.numpy as jnp
from jax import lax
from jax.experimental import pallas as pl
from jax.experimental.pallas import tpu as pltpu
from jax.experimental.pallas import tpu_sc as plsc

CLS_DIM = 1000000
EMB_DIM = 64
BATCH = 16384

NUM_CORES = 2
NUM_SUBCORES = 16
NUM_WORKERS = NUM_CORES * NUM_SUBCORES
B_PER_W = BATCH // NUM_WORKERS
LANES = 16
NCHUNK = B_PER_W // LANES


def _make_kernel():
    mesh = plsc.VectorSubcoreMesh(core_axis_name="c", subcore_axis_name="s")

    @functools.partial(
        pl.kernel,
        mesh=mesh,
        out_type=jax.ShapeDtypeStruct((BATCH * EMB_DIM // 128, 128), jnp.float32),
        scratch_types=[
            pltpu.VMEM((B_PER_W,), jnp.int32),
            pltpu.VMEM((2, LANES, 8, EMB_DIM), jnp.float32),
            pltpu.VMEM((2, LANES * EMB_DIM // 128, 128), jnp.float32),
            pltpu.SemaphoreType.DMA,
            pltpu.SemaphoreType.DMA,
            pltpu.SemaphoreType.DMA,
            pltpu.SemaphoreType.DMA,
        ],
        compiler_params=pltpu.CompilerParams(needs_layout_passes=False),
    )
    def gather_kernel(idx_hbm, tab_hbm, out_hbm,
                      idx_v, slots, outbuf, sem0, sem1, osem0, osem1):
        wid = lax.axis_index("s") * NUM_CORES + lax.axis_index("c")
        pltpu.sync_copy(idx_hbm.at[wid], idx_v)
        lanes = lax.iota(jnp.int32, LANES)
        sems = [sem0, sem1]
        osems = [osem0, osem1]
        orows = LANES * EMB_DIM // 128
        obase = wid * B_PER_W * EMB_DIM // 128

        def scalar_at(base16, l):
            v = idx_v[pl.ds(base16, LANES)]
            return jnp.sum(jnp.where(lanes == l, v, 0))

        def fire(c, par):
            for l in range(LANES):
                i = scalar_at(c * LANES, l)
                g = lax.shift_right_logical(i, 3)
                pltpu.async_copy(
                    tab_hbm.at[g], slots.at[par, l], sems[par]
                )

        def drain(d, par):
            for l in range(LANES):
                pltpu.make_async_copy(
                    tab_hbm.at[0], slots.at[par, l], sems[par]
                ).wait()
            for l in range(LANES):
                i = scalar_at(d * LANES, l)
                r = lax.bitwise_and(i, 7)
                row = l // 2
                colbase = (l % 2) * EMB_DIM
                for c4 in range(EMB_DIM // LANES):
                    outbuf[par, row, pl.ds(colbase + c4 * LANES, LANES)] = (
                        slots[par, l, r, pl.ds(c4 * LANES, LANES)]
                    )
            pltpu.async_copy(
                outbuf.at[par],
                out_hbm.at[pl.ds(pl.multiple_of(obase + d * orows, 8), orows)],
                osems[par],
            )

        def owait(par):
            pltpu.make_async_copy(
                out_hbm.at[pl.ds(0, orows)], outbuf.at[par], osems[par]
            ).wait()

        fire(0, 0)

        def body(c2, _):
            a = 2 * c2
            b = a + 1

            @pl.when(c2 >= 1)
            def _w0():
                owait(0)

            fire(b, 1)
            drain(a, 0)

            @pl.when(c2 >= 1)
            def _w1():
                owait(1)

            @pl.when(c2 < NCHUNK // 2 - 1)
            def _f1():
                fire(b + 1, 0)

            drain(b, 1)
            return _

        lax.fori_loop(0, NCHUNK // 2, body, None)
        owait(0)
        owait(1)

    return gather_kernel


_gather = _make_kernel()


@jax.jit
def kernel(batch, table):
    idx = batch.astype(jnp.int32).reshape(NUM_WORKERS, B_PER_W)
    tab3 = table.reshape(CLS_DIM // 8, 8, EMB_DIM)
    out = _gather(idx, tab3)
    return out.reshape(BATCH, EMB_DIM)


# speedup vs baseline: 2.7039x; 1.0318x over previous
import functools

import jax
import jax.numpy as jnp
from jax import lax
from jax.experimental import pallas as pl
from jax.experimental.pallas import tpu as pltpu
from jax.experimental.pallas import tpu_sc as plsc

CLS_DIM = 1000000
EMB_DIM = 64
BATCH = 16384

NUM_CORES = 2
NUM_SUBCORES = 16
NUM_WORKERS = NUM_CORES * NUM_SUBCORES
B_PER_W = BATCH // NUM_WORKERS
LANES = 16
NCHUNK = B_PER_W // LANES


def _make_kernel():
    mesh = plsc.VectorSubcoreMesh(core_axis_name="c", subcore_axis_name="s")

    @functools.partial(
        pl.kernel,
        mesh=mesh,
        out_type=jax.ShapeDtypeStruct((BATCH, EMB_DIM), jnp.float32),
        scratch_types=[
            pltpu.VMEM((B_PER_W,), jnp.int32),
            pltpu.VMEM((2, LANES, 8, EMB_DIM), jnp.float32),
            pltpu.VMEM((2, LANES, EMB_DIM), jnp.float32),
            pltpu.SemaphoreType.DMA,
            pltpu.SemaphoreType.DMA,
            pltpu.SemaphoreType.DMA,
            pltpu.SemaphoreType.DMA,
        ],
        compiler_params=pltpu.CompilerParams(needs_layout_passes=False),
    )
    def gather_kernel(idx_hbm, tab_hbm, out_hbm,
                      idx_v, slots, outbuf, sem0, sem1, osem0, osem1):
        wid = lax.axis_index("s") * NUM_CORES + lax.axis_index("c")
        pltpu.sync_copy(idx_hbm.at[wid], idx_v)
        lanes = lax.iota(jnp.int32, LANES)
        sems = [sem0, sem1]
        osems = [osem0, osem1]
        obase = wid * B_PER_W

        def scalar_at(base16, l):
            v = idx_v[pl.ds(base16, LANES)]
            return v[l]

        def fire(c, par):
            for l in range(LANES):
                i = scalar_at(c * LANES, l)
                g = lax.shift_right_logical(i, 3)
                pltpu.async_copy(
                    tab_hbm.at[g], slots.at[par, l], sems[par]
                )

        def drain(d, par):
            for l in range(LANES):
                pltpu.make_async_copy(
                    tab_hbm.at[0], slots.at[par, l], sems[par]
                ).wait()
            for l in range(LANES):
                i = scalar_at(d * LANES, l)
                r = lax.bitwise_and(i, 7)
                for c4 in range(EMB_DIM // LANES):
                    outbuf[par, l, pl.ds(c4 * LANES, LANES)] = (
                        slots[par, l, r, pl.ds(c4 * LANES, LANES)]
                    )
            pltpu.async_copy(
                outbuf.at[par],
                out_hbm.at[pl.ds(pl.multiple_of(obase + d * LANES, 8), LANES)],
                osems[par],
            )

        def owait(par):
            pltpu.make_async_copy(
                out_hbm.at[pl.ds(0, LANES)], outbuf.at[par], osems[par]
            ).wait()

        fire(0, 0)

        def body(c2, _):
            a = 2 * c2
            b = a + 1

            @pl.when(c2 >= 1)
            def _w0():
                owait(0)

            fire(b, 1)
            drain(a, 0)

            @pl.when(c2 >= 1)
            def _w1():
                owait(1)

            @pl.when(c2 < NCHUNK // 2 - 1)
            def _f1():
                fire(b + 1, 0)

            drain(b, 1)
            return _

        lax.fori_loop(0, NCHUNK // 2, body, None)
        owait(0)
        owait(1)

    return gather_kernel


_gather = _make_kernel()


@jax.jit
def kernel(batch, table):
    idx = batch.astype(jnp.int32).reshape(NUM_WORKERS, B_PER_W)
    tab3 = table.reshape(CLS_DIM // 8, 8, EMB_DIM)
    return _gather(idx, tab3)


# speedup vs baseline: 2.7525x; 1.0180x over previous
import functools

import jax
import jax.numpy as jnp
from jax import lax
from jax.experimental import pallas as pl
from jax.experimental.pallas import tpu as pltpu
from jax.experimental.pallas import tpu_sc as plsc

CLS_DIM = 1000000
EMB_DIM = 64
BATCH = 16384

NUM_CORES = 2
NUM_SUBCORES = 16
NUM_WORKERS = NUM_CORES * NUM_SUBCORES
B_PER_W = BATCH // NUM_WORKERS
LANES = 16
NCHUNK = B_PER_W // LANES


def _make_kernel():
    mesh = plsc.VectorSubcoreMesh(core_axis_name="c", subcore_axis_name="s")

    @functools.partial(
        pl.kernel,
        mesh=mesh,
        out_type=jax.ShapeDtypeStruct((BATCH, EMB_DIM), jnp.float32),
        scratch_types=[
            pltpu.VMEM((B_PER_W,), jnp.int32),
            pltpu.VMEM((4, LANES, 8, EMB_DIM), jnp.float32),
            pltpu.VMEM((4, LANES, EMB_DIM), jnp.float32),
            pltpu.SemaphoreType.DMA,
            pltpu.SemaphoreType.DMA,
            pltpu.SemaphoreType.DMA,
            pltpu.SemaphoreType.DMA,
            pltpu.SemaphoreType.DMA,
            pltpu.SemaphoreType.DMA,
            pltpu.SemaphoreType.DMA,
            pltpu.SemaphoreType.DMA,
        ],
        compiler_params=pltpu.CompilerParams(needs_layout_passes=False),
    )
    def gather_kernel(idx_hbm, tab_hbm, out_hbm, idx_v, slots, outbuf,
                      sem0, sem1, sem2, sem3, osem0, osem1, osem2, osem3):
        wid = lax.axis_index("s") * NUM_CORES + lax.axis_index("c")
        pltpu.sync_copy(idx_hbm.at[wid], idx_v)
        lanes = lax.iota(jnp.int32, LANES)
        sems = [sem0, sem1, sem2, sem3]
        osems = [osem0, osem1, osem2, osem3]
        obase = wid * B_PER_W

        def scalar_at(base16, l):
            v = idx_v[pl.ds(base16, LANES)]
            return v[l]

        def fire(c, par):
            for l in range(LANES):
                i = scalar_at(c * LANES, l)
                g = lax.shift_right_logical(i, 3)
                pltpu.async_copy(
                    tab_hbm.at[g], slots.at[par, l], sems[par]
                )

        def drain(d, par):
            pltpu.make_async_copy(
                tab_hbm.at[pl.ds(0, LANES)], slots.at[par], sems[par]
            ).wait()
            for l in range(LANES):
                i = scalar_at(d * LANES, l)
                r = lax.bitwise_and(i, 7)
                for c4 in range(EMB_DIM // LANES):
                    outbuf[par, l, pl.ds(c4 * LANES, LANES)] = (
                        slots[par, l, r, pl.ds(c4 * LANES, LANES)]
                    )
            pltpu.async_copy(
                outbuf.at[par],
                out_hbm.at[pl.ds(pl.multiple_of(obase + d * LANES, 8), LANES)],
                osems[par],
            )

        def owait(par):
            pltpu.make_async_copy(
                out_hbm.at[pl.ds(0, LANES)], outbuf.at[par], osems[par]
            ).wait()

        fire(0, 0)
        fire(1, 1)
        fire(2, 2)

        def body(c4, _):
            for j in range(4):
                c = 4 * c4 + j

                @pl.when(c4 >= 1)
                def _w():
                    owait(j)

                if j == 0:
                    fire(c + 3, 3)
                else:
                    @pl.when(c4 < NCHUNK // 4 - 1)
                    def _f():
                        fire(c + 3, (j + 3) % 4)

                drain(c, j)
            return _

        lax.fori_loop(0, NCHUNK // 4, body, None)
        for j in range(4):
            owait(j)

    return gather_kernel


_gather = _make_kernel()


@jax.jit
def kernel(batch, table):
    idx = batch.astype(jnp.int32).reshape(NUM_WORKERS, B_PER_W)
    tab3 = table.reshape(CLS_DIM // 8, 8, EMB_DIM)
    return _gather(idx, tab3)


# speedup vs baseline: 2.7548x; 1.0009x over previous
import functools

import jax
import jax.numpy as jnp
from jax import lax
from jax.experimental import pallas as pl
from jax.experimental.pallas import tpu as pltpu
from jax.experimental.pallas import tpu_sc as plsc

CLS_DIM = 1000000
EMB_DIM = 64
BATCH = 16384

NUM_CORES = 2
NUM_SUBCORES = 16
NUM_WORKERS = NUM_CORES * NUM_SUBCORES
B_PER_W = BATCH // NUM_WORKERS
LANES = 16
NCHUNK = B_PER_W // LANES


def _make_kernel():
    mesh = plsc.VectorSubcoreMesh(core_axis_name="c", subcore_axis_name="s")

    @functools.partial(
        pl.kernel,
        mesh=mesh,
        out_type=jax.ShapeDtypeStruct((BATCH, EMB_DIM), jnp.float32),
        scratch_types=[
            pltpu.VMEM((B_PER_W,), jnp.int32),
            pltpu.VMEM((4, LANES, 8, EMB_DIM), jnp.float32),
            pltpu.VMEM((4, LANES, EMB_DIM), jnp.float32),
            pltpu.SemaphoreType.DMA,
            pltpu.SemaphoreType.DMA,
            pltpu.SemaphoreType.DMA,
            pltpu.SemaphoreType.DMA,
            pltpu.SemaphoreType.DMA,
            pltpu.SemaphoreType.DMA,
            pltpu.SemaphoreType.DMA,
            pltpu.SemaphoreType.DMA,
        ],
        compiler_params=pltpu.CompilerParams(needs_layout_passes=False),
    )
    def gather_kernel(idx_hbm, tab_hbm, out_hbm, idx_v, slots, outbuf,
                      sem0, sem1, sem2, sem3, osem0, osem1, osem2, osem3):
        wid = lax.axis_index("s") * NUM_CORES + lax.axis_index("c")
        pltpu.sync_copy(idx_hbm.at[wid], idx_v)
        sems = [sem0, sem1, sem2, sem3]
        osems = [osem0, osem1, osem2, osem3]
        obase = wid * B_PER_W

        def scalar_at(base16, l):
            v = idx_v[pl.ds(base16, LANES)]
            return v[l]

        def fire(c, par):
            for l in range(LANES):
                i = scalar_at(c * LANES, l)
                g = lax.shift_right_logical(i, 3)
                pltpu.async_copy(
                    tab_hbm.at[g], slots.at[par, l], sems[par]
                )

        def drain(d, par):
            pltpu.make_async_copy(
                tab_hbm.at[pl.ds(0, LANES)], slots.at[par], sems[par]
            ).wait()
            for l in range(LANES):
                i = scalar_at(d * LANES, l)
                r = lax.bitwise_and(i, 7)
                for c4 in range(EMB_DIM // LANES):
                    outbuf[par, l, pl.ds(c4 * LANES, LANES)] = (
                        slots[par, l, r, pl.ds(c4 * LANES, LANES)]
                    )
            pltpu.async_copy(
                outbuf.at[par],
                out_hbm.at[pl.ds(pl.multiple_of(obase + d * LANES, 8), LANES)],
                osems[par],
            )

        def owait(par):
            pltpu.make_async_copy(
                out_hbm.at[pl.ds(0, LANES)], outbuf.at[par], osems[par]
            ).wait()

        fire(0, 0)
        fire(1, 1)
        fire(2, 2)

        def body(c4, _):
            for j in range(4):
                c = 4 * c4 + j

                @pl.when(c4 >= 1)
                def _w():
                    owait(j)

                if j == 0:
                    fire(c + 3, 3)
                else:
                    @pl.when(c4 < NCHUNK // 4 - 1)
                    def _f():
                        fire(c + 3, (j + 3) % 4)

                drain(c, j)
            return _

        lax.fori_loop(0, NCHUNK // 4, body, None)
        for j in range(4):
            owait(j)

    return gather_kernel


_gather = _make_kernel()


@jax.jit
def kernel(batch, table):
    idx = batch.astype(jnp.int32).reshape(NUM_WORKERS, B_PER_W)
    tab3 = table.reshape(CLS_DIM // 8, 8, EMB_DIM)
    return _gather(idx, tab3)
